# baseline (speedup 1.0000x reference)
import functools

import jax
import jax.numpy as jnp
from jax import lax
from jax.experimental import pallas as pl
from jax.experimental.pallas import tpu as pltpu
from jax.experimental.pallas import tpu_sc as plsc

N = 10000
E = 320000
D = 128
H = 128
C = 2

NC = 2
NS = 16
NW = NC * NS

_MESH = dict(core_axis_name="c", subcore_axis_name="s", num_cores=NC,
             num_subcores=NS)

ROWS_PER_TILE = 640
LAST_ROWS = N - 15 * ROWS_PER_TILE


CB = 128
NCHUNK = E // CB
ITERS = (NCHUNK + NW - 1) // NW


def _edge_agg_kernel(dw: int, tc_tiling: bool = True):
  @functools.partial(
      pl.kernel,
      out_type=jax.ShapeDtypeStruct((NC, N, dw), jnp.float32),
      mesh=plsc.VectorSubcoreMesh(**_MESH),
      compiler_params=pltpu.CompilerParams(use_tc_tiling_on_sc=tc_tiling),
      scratch_types=[
          pltpu.VMEM((4, CB), jnp.int32),
          pltpu.VMEM((4, CB), jnp.int32),
          pltpu.VMEM((2, CB, dw), jnp.float32),
          pltpu.VMEM_SHARED((N, dw), jnp.float32),
          pltpu.SemaphoreType.DMA((4,)),
          pltpu.SemaphoreType.DMA((2,)),
          pltpu.SemaphoreType.DMA((2,)),
      ],
  )
  def agg(h_hbm, edge_hbm, out_hbm, src_v, dst_v, rows_v,
          acc_sh, isem, gsem, ssem):
    c = lax.axis_index("c")
    s = lax.axis_index("s")
    wid = s * NC + c
    row0 = s * ROWS_PER_TILE

    def chunk_ix(i):
      return wid + i * NW

    def valid(i):
      return chunk_ix(i) < NCHUNK

    def issue_idx(i):
      slot = lax.rem(i, 4)
      base = chunk_ix(i) * CB
      pltpu.async_copy(edge_hbm.at[0, pl.ds(base, CB)], src_v.at[slot],
                       isem.at[slot])
      pltpu.async_copy(edge_hbm.at[1, pl.ds(base, CB)], dst_v.at[slot],
                       isem.at[slot])

    def wait_idx(i):
      slot = lax.rem(i, 4)
      pltpu.make_async_copy(edge_hbm.at[0, pl.ds(0, CB)], src_v.at[slot],
                            isem.at[slot]).wait()
      pltpu.make_async_copy(edge_hbm.at[1, pl.ds(0, CB)], dst_v.at[slot],
                            isem.at[slot]).wait()

    def issue_gather(i):
      slot = lax.rem(i, 4)
      p = lax.rem(i, 2)
      pltpu.async_copy(h_hbm.at[src_v.at[slot]], rows_v.at[p], gsem.at[p])

    def wait_gather(i):
      slot = lax.rem(i, 4)
      p = lax.rem(i, 2)
      pltpu.make_async_copy(h_hbm.at[src_v.at[slot]], rows_v.at[p],
                            gsem.at[p]).wait()

    def issue_scatter(i):
      slot = lax.rem(i, 4)
      p = lax.rem(i, 2)
      pltpu.async_copy(rows_v.at[p], acc_sh.at[dst_v.at[slot]], ssem.at[p],
                       add=True)

    def wait_scatter(i):
      slot = lax.rem(i, 4)
      p = lax.rem(i, 2)
      pltpu.make_async_copy(rows_v.at[p], acc_sh.at[dst_v.at[slot]],
                            ssem.at[p]).wait()

    def zrow(r, carry):
      for j in range(dw // 16):
        rows_v[0, r, pl.ds(j * 16, 16)] = jnp.zeros((16,), jnp.float32)
      return carry

    lax.fori_loop(0, CB, zrow, 0)

    @pl.when(s < NS - 1)
    def _():
      for k in range(ROWS_PER_TILE // CB):
        pltpu.sync_copy(rows_v.at[0],
                        acc_sh.at[pl.ds(row0 + k * CB, CB)])

    @pl.when(s == NS - 1)
    def _():
      for k in range(LAST_ROWS // CB):
        pltpu.sync_copy(rows_v.at[0],
                        acc_sh.at[pl.ds(row0 + k * CB, CB)])
      rem = LAST_ROWS % CB
      if rem:
        pltpu.sync_copy(rows_v.at[0, pl.ds(0, rem)],
                        acc_sh.at[pl.ds(row0 + (LAST_ROWS // CB) * CB, rem)])

    plsc.subcore_barrier()

    @pl.when(valid(0))
    def _():
      issue_idx(0)

    @pl.when(valid(1))
    def _():
      issue_idx(1)

    def body(i, carry):
      @pl.when((i >= 2) & valid(i - 2))
      def _():
        wait_scatter(i - 2)

      @pl.when(valid(i + 2))
      def _():
        issue_idx(i + 2)

      @pl.when(valid(i))
      def _():
        wait_idx(i)
        issue_gather(i)

      @pl.when((i >= 1) & valid(i - 1))
      def _():
        wait_gather(i - 1)
        issue_scatter(i - 1)

      return carry

    lax.fori_loop(0, ITERS + 1, body, 0)

    @pl.when(valid(ITERS - 1))
    def _():
      wait_scatter(ITERS - 1)

    plsc.subcore_barrier()

    @pl.when(s < NS - 1)
    def _():
      pltpu.sync_copy(acc_sh.at[pl.ds(row0, ROWS_PER_TILE)],
                      out_hbm.at[c, pl.ds(row0, ROWS_PER_TILE)])

    @pl.when(s == NS - 1)
    def _():
      pltpu.sync_copy(acc_sh.at[pl.ds(row0, LAST_ROWS)],
                      out_hbm.at[c, pl.ds(row0, LAST_ROWS)])

  return agg


_agg128 = _edge_agg_kernel(128)
_agg16 = _edge_agg_kernel(16, tc_tiling=False)


@functools.partial(
    pl.kernel,
    out_type=jax.ShapeDtypeStruct((NC, 2, N), jnp.float32),
    mesh=plsc.VectorSubcoreMesh(**_MESH),
    scratch_types=[
        pltpu.VMEM((4, CB), jnp.int32),
        pltpu.VMEM((4, CB), jnp.int32),
        pltpu.VMEM((CB,), jnp.float32),
        pltpu.VMEM((CB,), jnp.float32),
        pltpu.VMEM_SHARED((N,), jnp.float32),
        pltpu.VMEM_SHARED((N,), jnp.float32),
        pltpu.SemaphoreType.DMA((4,)),
        pltpu.SemaphoreType.DMA((2,)),
    ],
)
def _deg_kernel(edge_hbm, out_hbm, src_v, dst_v, ones_v, zero_v,
                dego_sh, degi_sh, isem, ssem):
  c = lax.axis_index("c")
  s = lax.axis_index("s")
  wid = s * NC + c
  for j in range(CB // 16):
    ones_v[pl.ds(j * 16, 16)] = jnp.ones((16,), jnp.float32)
    zero_v[pl.ds(j * 16, 16)] = jnp.zeros((16,), jnp.float32)

  def chunk_ix(i):
    return wid + i * NW

  def valid(i):
    return chunk_ix(i) < NCHUNK

  def issue_idx(i):
    slot = lax.rem(i, 4)
    base = chunk_ix(i) * CB
    pltpu.async_copy(edge_hbm.at[0, pl.ds(base, CB)], src_v.at[slot],
                     isem.at[slot])
    pltpu.async_copy(edge_hbm.at[1, pl.ds(base, CB)], dst_v.at[slot],
                     isem.at[slot])

  def wait_idx(i):
    slot = lax.rem(i, 4)
    pltpu.make_async_copy(edge_hbm.at[0, pl.ds(0, CB)], src_v.at[slot],
                          isem.at[slot]).wait()
    pltpu.make_async_copy(edge_hbm.at[1, pl.ds(0, CB)], dst_v.at[slot],
                          isem.at[slot]).wait()

  def issue_scatter(i):
    slot = lax.rem(i, 4)
    p = lax.rem(i, 2)
    pltpu.async_copy(ones_v, dego_sh.at[src_v.at[slot]], ssem.at[p],
                     add=True)
    pltpu.async_copy(ones_v, degi_sh.at[dst_v.at[slot]], ssem.at[p],
                     add=True)

  def wait_scatter(i):
    slot = lax.rem(i, 4)
    p = lax.rem(i, 2)
    pltpu.make_async_copy(ones_v, dego_sh.at[src_v.at[slot]],
                          ssem.at[p]).wait()
    pltpu.make_async_copy(ones_v, degi_sh.at[dst_v.at[slot]],
                          ssem.at[p]).wait()

  @pl.when(s < NS - 1)
  def _():
    for k in range(ROWS_PER_TILE // CB):
      pltpu.sync_copy(zero_v, dego_sh.at[pl.ds(s * ROWS_PER_TILE + k * CB, CB)])
      pltpu.sync_copy(zero_v, degi_sh.at[pl.ds(s * ROWS_PER_TILE + k * CB, CB)])

  @pl.when(s == NS - 1)
  def _():
    base0 = (NS - 1) * ROWS_PER_TILE
    for k in range(LAST_ROWS // CB):
      pltpu.sync_copy(zero_v, dego_sh.at[pl.ds(base0 + k * CB, CB)])
      pltpu.sync_copy(zero_v, degi_sh.at[pl.ds(base0 + k * CB, CB)])
    rem = LAST_ROWS % CB
    if rem:
      pltpu.sync_copy(zero_v.at[pl.ds(0, rem)],
                      dego_sh.at[pl.ds(base0 + (LAST_ROWS // CB) * CB, rem)])
      pltpu.sync_copy(zero_v.at[pl.ds(0, rem)],
                      degi_sh.at[pl.ds(base0 + (LAST_ROWS // CB) * CB, rem)])

  plsc.subcore_barrier()

  @pl.when(valid(0))
  def _():
    issue_idx(0)

  @pl.when(valid(1))
  def _():
    issue_idx(1)

  def body(i, carry):
    @pl.when((i >= 2) & valid(i - 2))
    def _():
      wait_scatter(i - 2)

    @pl.when(valid(i + 2))
    def _():
      issue_idx(i + 2)

    @pl.when(valid(i))
    def _():
      wait_idx(i)
      issue_scatter(i)

    return carry

  lax.fori_loop(0, ITERS, body, 0)

  @pl.when(valid(ITERS - 2))
  def _():
    wait_scatter(ITERS - 2)

  @pl.when(valid(ITERS - 1))
  def _():
    wait_scatter(ITERS - 1)

  plsc.subcore_barrier()

  @pl.when(s == 0)
  def _():
    pltpu.sync_copy(dego_sh, out_hbm.at[c, 0])
    pltpu.sync_copy(degi_sh, out_hbm.at[c, 1])


def _tc0_body(x_ref, w1_ref, xw_ref):
  xw_ref[...] = jnp.dot(x_ref[...], w1_ref[...],
                        preferred_element_type=jnp.float32)


def _tc1_body(xw_ref, degp_ref, h1_ref, norms_ref):
  dp = degp_ref[...]
  deg_out = dp[:, 0:1] + dp[:, 2:3]
  deg_in = dp[:, 1:2] + dp[:, 3:4]
  ns = lax.rsqrt(jnp.maximum(deg_out, 1.0))
  nd = lax.rsqrt(jnp.maximum(deg_in, 1.0))
  h1_ref[...] = xw_ref[...] * ns
  norms_ref[...] = jnp.concatenate([ns, nd], axis=1)


def _tc2_body(aggp_ref, norms_ref, b1_ref, w2_ref, h2_ref):
  nrm = norms_ref[...]
  o1 = jnp.maximum((aggp_ref[0] + aggp_ref[1]) * nrm[:, 1:2] + b1_ref[...],
                   0.0)
  h2_ref[...] = jnp.dot(o1 * nrm[:, 0:1], w2_ref[...],
                        preferred_element_type=jnp.float32)


def _tc3_body(qp_ref, norms_ref, b2_ref, out_ref):
  nrm = norms_ref[...]
  q = (qp_ref[0] + qp_ref[1]) * nrm[:, 1:2]
  out_ref[...] = q[:, 0:C] + b2_ref[...]


@jax.jit
def kernel(x, edge_index, W1, b1, W2, b2):
  degp = _deg_kernel(edge_index)
  degp4 = degp.reshape(NC * 2, N).transpose(1, 0)

  xw = pl.pallas_call(
      _tc0_body,
      out_shape=jax.ShapeDtypeStruct((N, H), jnp.float32),
  )(x, W1)

  h1, norms = pl.pallas_call(
      _tc1_body,
      out_shape=[
          jax.ShapeDtypeStruct((N, H), jnp.float32),
          jax.ShapeDtypeStruct((N, 2), jnp.float32),
      ],
  )(xw, degp4)

  aggp = _agg128(h1, edge_index)

  w2p = jnp.pad(W2, ((0, 0), (0, 16 - C)))
  h2 = pl.pallas_call(
      _tc2_body,
      out_shape=jax.ShapeDtypeStruct((N, 16), jnp.float32),
  )(aggp, norms, b1.reshape(1, H), w2p)

  qp = _agg16(h2, edge_index)

  out = pl.pallas_call(
      _tc3_body,
      out_shape=jax.ShapeDtypeStruct((N, C), jnp.float32),
  )(qp, norms, b2.reshape(1, C))
  return out


# speedup vs baseline: 1.0105x; 1.0105x over previous
import functools

import jax
import jax.numpy as jnp
from jax import lax
from jax.experimental import pallas as pl
from jax.experimental.pallas import tpu as pltpu
from jax.experimental.pallas import tpu_sc as plsc

N = 10000
E = 320000
D = 128
H = 128
C = 2

NC = 2
NS = 16
NW = NC * NS

_MESH = dict(core_axis_name="c", subcore_axis_name="s", num_cores=NC,
             num_subcores=NS)

ROWS_PER_TILE = 640
LAST_ROWS = N - 15 * ROWS_PER_TILE


CB = 128
NCHUNK = E // CB
ITERS = (NCHUNK + NW - 1) // NW


def _edge_agg_kernel(dw: int, tc_tiling: bool = True):
  @functools.partial(
      pl.kernel,
      out_type=jax.ShapeDtypeStruct((NC, N, dw), jnp.float32),
      mesh=plsc.VectorSubcoreMesh(**_MESH),
      compiler_params=pltpu.CompilerParams(use_tc_tiling_on_sc=tc_tiling),
      scratch_types=[
          pltpu.VMEM((4, CB), jnp.int32),
          pltpu.VMEM((4, CB), jnp.int32),
          pltpu.VMEM((2, CB, dw), jnp.float32),
          pltpu.VMEM_SHARED((N, dw), jnp.float32),
          pltpu.SemaphoreType.DMA((4,)),
          pltpu.SemaphoreType.DMA((2,)),
          pltpu.SemaphoreType.DMA((2,)),
      ],
  )
  def agg(h_hbm, edge_hbm, out_hbm, src_v, dst_v, rows_v,
          acc_sh, isem, gsem, ssem):
    c = lax.axis_index("c")
    s = lax.axis_index("s")
    wid = s * NC + c
    row0 = s * ROWS_PER_TILE

    def chunk_ix(i):
      return wid + i * NW

    def valid(i):
      return chunk_ix(i) < NCHUNK

    def issue_idx(i):
      slot = lax.rem(i, 4)
      base = chunk_ix(i) * CB
      pltpu.async_copy(edge_hbm.at[0, pl.ds(base, CB)], src_v.at[slot],
                       isem.at[slot])
      pltpu.async_copy(edge_hbm.at[1, pl.ds(base, CB)], dst_v.at[slot],
                       isem.at[slot])

    def wait_idx(i):
      slot = lax.rem(i, 4)
      pltpu.make_async_copy(edge_hbm.at[0, pl.ds(0, CB)], src_v.at[slot],
                            isem.at[slot]).wait()
      pltpu.make_async_copy(edge_hbm.at[1, pl.ds(0, CB)], dst_v.at[slot],
                            isem.at[slot]).wait()

    def issue_gather(i):
      slot = lax.rem(i, 4)
      p = lax.rem(i, 2)
      pltpu.async_copy(h_hbm.at[src_v.at[slot]], rows_v.at[p], gsem.at[p])

    def wait_gather(i):
      slot = lax.rem(i, 4)
      p = lax.rem(i, 2)
      pltpu.make_async_copy(h_hbm.at[src_v.at[slot]], rows_v.at[p],
                            gsem.at[p]).wait()

    def issue_scatter(i):
      slot = lax.rem(i, 4)
      p = lax.rem(i, 2)
      pltpu.async_copy(rows_v.at[p], acc_sh.at[dst_v.at[slot]], ssem.at[p],
                       add=True)

    def wait_scatter(i):
      slot = lax.rem(i, 4)
      p = lax.rem(i, 2)
      pltpu.make_async_copy(rows_v.at[p], acc_sh.at[dst_v.at[slot]],
                            ssem.at[p]).wait()

    def zrow(r, carry):
      for j in range(dw // 16):
        rows_v[0, r, pl.ds(j * 16, 16)] = jnp.zeros((16,), jnp.float32)
      return carry

    lax.fori_loop(0, CB, zrow, 0)

    @pl.when(s < NS - 1)
    def _():
      for k in range(ROWS_PER_TILE // CB):
        pltpu.sync_copy(rows_v.at[0],
                        acc_sh.at[pl.ds(row0 + k * CB, CB)])

    @pl.when(s == NS - 1)
    def _():
      for k in range(LAST_ROWS // CB):
        pltpu.sync_copy(rows_v.at[0],
                        acc_sh.at[pl.ds(row0 + k * CB, CB)])
      rem = LAST_ROWS % CB
      if rem:
        pltpu.sync_copy(rows_v.at[0, pl.ds(0, rem)],
                        acc_sh.at[pl.ds(row0 + (LAST_ROWS // CB) * CB, rem)])

    plsc.subcore_barrier()

    @pl.when(valid(0))
    def _():
      issue_idx(0)

    @pl.when(valid(1))
    def _():
      issue_idx(1)

    def body(i, carry):
      @pl.when((i >= 2) & valid(i - 2))
      def _():
        wait_scatter(i - 2)

      @pl.when(valid(i + 2))
      def _():
        issue_idx(i + 2)

      @pl.when(valid(i))
      def _():
        wait_idx(i)
        issue_gather(i)

      @pl.when((i >= 1) & valid(i - 1))
      def _():
        wait_gather(i - 1)
        issue_scatter(i - 1)

      return carry

    lax.fori_loop(0, ITERS + 1, body, 0)

    @pl.when(valid(ITERS - 1))
    def _():
      wait_scatter(ITERS - 1)

    plsc.subcore_barrier()

    @pl.when(s < NS - 1)
    def _():
      pltpu.sync_copy(acc_sh.at[pl.ds(row0, ROWS_PER_TILE)],
                      out_hbm.at[c, pl.ds(row0, ROWS_PER_TILE)])

    @pl.when(s == NS - 1)
    def _():
      pltpu.sync_copy(acc_sh.at[pl.ds(row0, LAST_ROWS)],
                      out_hbm.at[c, pl.ds(row0, LAST_ROWS)])

  return agg


_agg128 = _edge_agg_kernel(128)
_agg16 = _edge_agg_kernel(16, tc_tiling=False)


@functools.partial(
    pl.kernel,
    out_type=jax.ShapeDtypeStruct((NC, 2, N), jnp.float32),
    mesh=plsc.VectorSubcoreMesh(**_MESH),
    scratch_types=[
        pltpu.VMEM((4, CB), jnp.int32),
        pltpu.VMEM((4, CB), jnp.int32),
        pltpu.VMEM((CB,), jnp.float32),
        pltpu.VMEM((CB,), jnp.float32),
        pltpu.VMEM_SHARED((N,), jnp.float32),
        pltpu.VMEM_SHARED((N,), jnp.float32),
        pltpu.SemaphoreType.DMA((4,)),
        pltpu.SemaphoreType.DMA((2,)),
    ],
)
def _deg_kernel(edge_hbm, out_hbm, src_v, dst_v, ones_v, zero_v,
                dego_sh, degi_sh, isem, ssem):
  c = lax.axis_index("c")
  s = lax.axis_index("s")
  wid = s * NC + c
  for j in range(CB // 16):
    ones_v[pl.ds(j * 16, 16)] = jnp.ones((16,), jnp.float32)
    zero_v[pl.ds(j * 16, 16)] = jnp.zeros((16,), jnp.float32)

  def chunk_ix(i):
    return wid + i * NW

  def valid(i):
    return chunk_ix(i) < NCHUNK

  def issue_idx(i):
    slot = lax.rem(i, 4)
    base = chunk_ix(i) * CB
    pltpu.async_copy(edge_hbm.at[0, pl.ds(base, CB)], src_v.at[slot],
                     isem.at[slot])
    pltpu.async_copy(edge_hbm.at[1, pl.ds(base, CB)], dst_v.at[slot],
                     isem.at[slot])

  def wait_idx(i):
    slot = lax.rem(i, 4)
    pltpu.make_async_copy(edge_hbm.at[0, pl.ds(0, CB)], src_v.at[slot],
                          isem.at[slot]).wait()
    pltpu.make_async_copy(edge_hbm.at[1, pl.ds(0, CB)], dst_v.at[slot],
                          isem.at[slot]).wait()

  def issue_scatter(i):
    slot = lax.rem(i, 4)
    p = lax.rem(i, 2)
    pltpu.async_copy(ones_v, dego_sh.at[src_v.at[slot]], ssem.at[p],
                     add=True)
    pltpu.async_copy(ones_v, degi_sh.at[dst_v.at[slot]], ssem.at[p],
                     add=True)

  def wait_scatter(i):
    slot = lax.rem(i, 4)
    p = lax.rem(i, 2)
    pltpu.make_async_copy(ones_v, dego_sh.at[src_v.at[slot]],
                          ssem.at[p]).wait()
    pltpu.make_async_copy(ones_v, degi_sh.at[dst_v.at[slot]],
                          ssem.at[p]).wait()

  @pl.when(s < NS - 1)
  def _():
    for k in range(ROWS_PER_TILE // CB):
      pltpu.sync_copy(zero_v, dego_sh.at[pl.ds(s * ROWS_PER_TILE + k * CB, CB)])
      pltpu.sync_copy(zero_v, degi_sh.at[pl.ds(s * ROWS_PER_TILE + k * CB, CB)])

  @pl.when(s == NS - 1)
  def _():
    base0 = (NS - 1) * ROWS_PER_TILE
    for k in range(LAST_ROWS // CB):
      pltpu.sync_copy(zero_v, dego_sh.at[pl.ds(base0 + k * CB, CB)])
      pltpu.sync_copy(zero_v, degi_sh.at[pl.ds(base0 + k * CB, CB)])
    rem = LAST_ROWS % CB
    if rem:
      pltpu.sync_copy(zero_v.at[pl.ds(0, rem)],
                      dego_sh.at[pl.ds(base0 + (LAST_ROWS // CB) * CB, rem)])
      pltpu.sync_copy(zero_v.at[pl.ds(0, rem)],
                      degi_sh.at[pl.ds(base0 + (LAST_ROWS // CB) * CB, rem)])

  plsc.subcore_barrier()

  @pl.when(valid(0))
  def _():
    issue_idx(0)

  @pl.when(valid(1))
  def _():
    issue_idx(1)

  def body(i, carry):
    @pl.when((i >= 2) & valid(i - 2))
    def _():
      wait_scatter(i - 2)

    @pl.when(valid(i + 2))
    def _():
      issue_idx(i + 2)

    @pl.when(valid(i))
    def _():
      wait_idx(i)
      issue_scatter(i)

    return carry

  lax.fori_loop(0, ITERS, body, 0)

  @pl.when(valid(ITERS - 2))
  def _():
    wait_scatter(ITERS - 2)

  @pl.when(valid(ITERS - 1))
  def _():
    wait_scatter(ITERS - 1)

  plsc.subcore_barrier()

  @pl.when(s == 0)
  def _():
    pltpu.sync_copy(dego_sh, out_hbm.at[c, 0])
    pltpu.sync_copy(degi_sh, out_hbm.at[c, 1])


def _tc1_body(x_ref, degp_ref, w1_ref, h1_ref, norms_ref):
  dp = degp_ref[...]
  deg_out = dp[:, 0:1] + dp[:, 2:3]
  deg_in = dp[:, 1:2] + dp[:, 3:4]
  ns = lax.rsqrt(jnp.maximum(deg_out, 1.0))
  nd = lax.rsqrt(jnp.maximum(deg_in, 1.0))
  h1_ref[...] = jnp.dot(x_ref[...] * ns, w1_ref[...],
                        preferred_element_type=jnp.float32)
  norms_ref[...] = jnp.concatenate([ns, nd], axis=1)


def _tc2_body(aggp_ref, norms_ref, b1_ref, w2_ref, h2_ref):
  nrm = norms_ref[...]
  o1 = jnp.maximum((aggp_ref[0] + aggp_ref[1]) * nrm[:, 1:2] + b1_ref[...],
                   0.0)
  h2_ref[...] = jnp.dot(o1 * nrm[:, 0:1], w2_ref[...],
                        preferred_element_type=jnp.float32)


def _tc3_body(qp_ref, norms_ref, b2_ref, out_ref):
  nrm = norms_ref[...]
  q = (qp_ref[0] + qp_ref[1]) * nrm[:, 1:2]
  out_ref[...] = q[:, 0:C] + b2_ref[...]


@jax.jit
def kernel(x, edge_index, W1, b1, W2, b2):
  degp = _deg_kernel(edge_index)
  degp4 = degp.reshape(NC * 2, N).transpose(1, 0)

  h1, norms = pl.pallas_call(
      _tc1_body,
      out_shape=[
          jax.ShapeDtypeStruct((N, H), jnp.float32),
          jax.ShapeDtypeStruct((N, 2), jnp.float32),
      ],
  )(x, degp4, W1)

  aggp = _agg128(h1, edge_index)

  w2p = jnp.pad(W2, ((0, 0), (0, 16 - C)))
  h2 = pl.pallas_call(
      _tc2_body,
      out_shape=jax.ShapeDtypeStruct((N, 16), jnp.float32),
  )(aggp, norms, b1.reshape(1, H), w2p)

  qp = _agg16(h2, edge_index)

  out = pl.pallas_call(
      _tc3_body,
      out_shape=jax.ShapeDtypeStruct((N, C), jnp.float32),
  )(qp, norms, b2.reshape(1, C))
  return out


# speedup vs baseline: 1.0662x; 1.0550x over previous
import functools

import jax
import jax.numpy as jnp
from jax import lax
from jax.experimental import pallas as pl
from jax.experimental.pallas import tpu as pltpu
from jax.experimental.pallas import tpu_sc as plsc

N = 10000
E = 320000
D = 128
H = 128
C = 2

NC = 2
NS = 16
NW = NC * NS

_MESH = dict(core_axis_name="c", subcore_axis_name="s", num_cores=NC,
             num_subcores=NS)

ROWS_PER_TILE = 640
LAST_ROWS = N - 15 * ROWS_PER_TILE


CB = 128
NCHUNK = E // CB
ITERS = (NCHUNK + NW - 1) // NW


def _edge_agg_kernel(dw: int, tc_tiling: bool = True, sb: int = 1):
  nstep = E // (CB * sb)
  iters = (nstep + NW - 1) // NW

  @functools.partial(
      pl.kernel,
      out_type=jax.ShapeDtypeStruct((NC, N, dw), jnp.float32),
      mesh=plsc.VectorSubcoreMesh(**_MESH),
      compiler_params=pltpu.CompilerParams(use_tc_tiling_on_sc=tc_tiling),
      scratch_types=[
          pltpu.VMEM((4, sb, CB), jnp.int32),
          pltpu.VMEM((4, sb, CB), jnp.int32),
          pltpu.VMEM((2, sb, CB, dw), jnp.float32),
          pltpu.VMEM_SHARED((N, dw), jnp.float32),
          pltpu.SemaphoreType.DMA((4,)),
          pltpu.SemaphoreType.DMA((2,)),
          pltpu.SemaphoreType.DMA((2,)),
      ],
  )
  def agg(h_hbm, edge_hbm, out_hbm, src_v, dst_v, rows_v,
          acc_sh, isem, gsem, ssem):
    c = lax.axis_index("c")
    s = lax.axis_index("s")
    wid = s * NC + c
    row0 = s * ROWS_PER_TILE

    def chunk_ix(i):
      return wid + i * NW

    def valid(i):
      return chunk_ix(i) < nstep

    def issue_idx(i):
      slot = lax.rem(i, 4)
      for j in range(sb):
        base = (chunk_ix(i) * sb + j) * CB
        pltpu.async_copy(edge_hbm.at[0, pl.ds(base, CB)],
                         src_v.at[slot, j], isem.at[slot])
        pltpu.async_copy(edge_hbm.at[1, pl.ds(base, CB)],
                         dst_v.at[slot, j], isem.at[slot])

    def wait_idx(i):
      slot = lax.rem(i, 4)
      for j in range(sb):
        pltpu.make_async_copy(edge_hbm.at[0, pl.ds(0, CB)],
                              src_v.at[slot, j], isem.at[slot]).wait()
        pltpu.make_async_copy(edge_hbm.at[1, pl.ds(0, CB)],
                              dst_v.at[slot, j], isem.at[slot]).wait()

    def issue_gather(i):
      slot = lax.rem(i, 4)
      p = lax.rem(i, 2)
      for j in range(sb):
        pltpu.async_copy(h_hbm.at[src_v.at[slot, j]], rows_v.at[p, j],
                         gsem.at[p])

    def wait_gather(i):
      slot = lax.rem(i, 4)
      p = lax.rem(i, 2)
      for j in range(sb):
        pltpu.make_async_copy(h_hbm.at[src_v.at[slot, j]], rows_v.at[p, j],
                              gsem.at[p]).wait()

    def issue_scatter(i):
      slot = lax.rem(i, 4)
      p = lax.rem(i, 2)
      for j in range(sb):
        pltpu.async_copy(rows_v.at[p, j], acc_sh.at[dst_v.at[slot, j]],
                         ssem.at[p], add=True)

    def wait_scatter(i):
      slot = lax.rem(i, 4)
      p = lax.rem(i, 2)
      for j in range(sb):
        pltpu.make_async_copy(rows_v.at[p, j], acc_sh.at[dst_v.at[slot, j]],
                              ssem.at[p]).wait()

    @pl.when(valid(0))
    def _():
      issue_idx(0)

    @pl.when(valid(1))
    def _():
      issue_idx(1)

    def zrow(r, carry):
      for j in range(dw // 16):
        rows_v[0, 0, r, pl.ds(j * 16, 16)] = jnp.zeros((16,), jnp.float32)
      return carry

    lax.fori_loop(0, CB, zrow, 0)

    @pl.when(s < NS - 1)
    def _():
      for k in range(ROWS_PER_TILE // CB):
        pltpu.sync_copy(rows_v.at[0, 0],
                        acc_sh.at[pl.ds(row0 + k * CB, CB)])

    @pl.when(s == NS - 1)
    def _():
      for k in range(LAST_ROWS // CB):
        pltpu.sync_copy(rows_v.at[0, 0],
                        acc_sh.at[pl.ds(row0 + k * CB, CB)])
      rem = LAST_ROWS % CB
      if rem:
        pltpu.sync_copy(rows_v.at[0, 0, pl.ds(0, rem)],
                        acc_sh.at[pl.ds(row0 + (LAST_ROWS // CB) * CB, rem)])

    plsc.subcore_barrier()

    def body(i, carry):
      @pl.when((i >= 2) & valid(i - 2))
      def _():
        wait_scatter(i - 2)

      @pl.when(valid(i + 2))
      def _():
        issue_idx(i + 2)

      @pl.when(valid(i))
      def _():
        wait_idx(i)
        issue_gather(i)

      @pl.when((i >= 1) & valid(i - 1))
      def _():
        wait_gather(i - 1)
        issue_scatter(i - 1)

      return carry

    lax.fori_loop(0, iters + 1, body, 0)

    @pl.when(valid(iters - 1))
    def _():
      wait_scatter(iters - 1)

    plsc.subcore_barrier()

    @pl.when(s < NS - 1)
    def _():
      pltpu.sync_copy(acc_sh.at[pl.ds(row0, ROWS_PER_TILE)],
                      out_hbm.at[c, pl.ds(row0, ROWS_PER_TILE)])

    @pl.when(s == NS - 1)
    def _():
      pltpu.sync_copy(acc_sh.at[pl.ds(row0, LAST_ROWS)],
                      out_hbm.at[c, pl.ds(row0, LAST_ROWS)])

  return agg


_agg128 = _edge_agg_kernel(128)
_agg16 = _edge_agg_kernel(16, tc_tiling=False, sb=2)


@functools.partial(
    pl.kernel,
    out_type=jax.ShapeDtypeStruct((NC, 2, N), jnp.float32),
    mesh=plsc.VectorSubcoreMesh(**_MESH),
    scratch_types=[
        pltpu.VMEM((4, CB), jnp.int32),
        pltpu.VMEM((4, CB), jnp.int32),
        pltpu.VMEM((CB,), jnp.float32),
        pltpu.VMEM((CB,), jnp.float32),
        pltpu.VMEM_SHARED((N,), jnp.float32),
        pltpu.VMEM_SHARED((N,), jnp.float32),
        pltpu.SemaphoreType.DMA((4,)),
        pltpu.SemaphoreType.DMA((2,)),
    ],
)
def _deg_kernel(edge_hbm, out_hbm, src_v, dst_v, ones_v, zero_v,
                dego_sh, degi_sh, isem, ssem):
  c = lax.axis_index("c")
  s = lax.axis_index("s")
  wid = s * NC + c
  for j in range(CB // 16):
    ones_v[pl.ds(j * 16, 16)] = jnp.ones((16,), jnp.float32)
    zero_v[pl.ds(j * 16, 16)] = jnp.zeros((16,), jnp.float32)

  def chunk_ix(i):
    return wid + i * NW

  def valid(i):
    return chunk_ix(i) < NCHUNK

  def issue_idx(i):
    slot = lax.rem(i, 4)
    base = chunk_ix(i) * CB
    pltpu.async_copy(edge_hbm.at[0, pl.ds(base, CB)], src_v.at[slot],
                     isem.at[slot])
    pltpu.async_copy(edge_hbm.at[1, pl.ds(base, CB)], dst_v.at[slot],
                     isem.at[slot])

  def wait_idx(i):
    slot = lax.rem(i, 4)
    pltpu.make_async_copy(edge_hbm.at[0, pl.ds(0, CB)], src_v.at[slot],
                          isem.at[slot]).wait()
    pltpu.make_async_copy(edge_hbm.at[1, pl.ds(0, CB)], dst_v.at[slot],
                          isem.at[slot]).wait()

  def issue_scatter(i):
    slot = lax.rem(i, 4)
    p = lax.rem(i, 2)
    pltpu.async_copy(ones_v, dego_sh.at[src_v.at[slot]], ssem.at[p],
                     add=True)
    pltpu.async_copy(ones_v, degi_sh.at[dst_v.at[slot]], ssem.at[p],
                     add=True)

  def wait_scatter(i):
    slot = lax.rem(i, 4)
    p = lax.rem(i, 2)
    pltpu.make_async_copy(ones_v, dego_sh.at[src_v.at[slot]],
                          ssem.at[p]).wait()
    pltpu.make_async_copy(ones_v, degi_sh.at[dst_v.at[slot]],
                          ssem.at[p]).wait()

  @pl.when(s < NS - 1)
  def _():
    for k in range(ROWS_PER_TILE // CB):
      pltpu.sync_copy(zero_v, dego_sh.at[pl.ds(s * ROWS_PER_TILE + k * CB, CB)])
      pltpu.sync_copy(zero_v, degi_sh.at[pl.ds(s * ROWS_PER_TILE + k * CB, CB)])

  @pl.when(s == NS - 1)
  def _():
    base0 = (NS - 1) * ROWS_PER_TILE
    for k in range(LAST_ROWS // CB):
      pltpu.sync_copy(zero_v, dego_sh.at[pl.ds(base0 + k * CB, CB)])
      pltpu.sync_copy(zero_v, degi_sh.at[pl.ds(base0 + k * CB, CB)])
    rem = LAST_ROWS % CB
    if rem:
      pltpu.sync_copy(zero_v.at[pl.ds(0, rem)],
                      dego_sh.at[pl.ds(base0 + (LAST_ROWS // CB) * CB, rem)])
      pltpu.sync_copy(zero_v.at[pl.ds(0, rem)],
                      degi_sh.at[pl.ds(base0 + (LAST_ROWS // CB) * CB, rem)])

  plsc.subcore_barrier()

  @pl.when(valid(0))
  def _():
    issue_idx(0)

  @pl.when(valid(1))
  def _():
    issue_idx(1)

  def body(i, carry):
    @pl.when((i >= 2) & valid(i - 2))
    def _():
      wait_scatter(i - 2)

    @pl.when(valid(i + 2))
    def _():
      issue_idx(i + 2)

    @pl.when(valid(i))
    def _():
      wait_idx(i)
      issue_scatter(i)

    return carry

  lax.fori_loop(0, ITERS, body, 0)

  @pl.when(valid(ITERS - 2))
  def _():
    wait_scatter(ITERS - 2)

  @pl.when(valid(ITERS - 1))
  def _():
    wait_scatter(ITERS - 1)

  plsc.subcore_barrier()

  @pl.when(s == 0)
  def _():
    pltpu.sync_copy(dego_sh, out_hbm.at[c, 0])
    pltpu.sync_copy(degi_sh, out_hbm.at[c, 1])


def _tc1_body(x_ref, degp_ref, w1_ref, h1_ref, norms_ref):
  dp = degp_ref[...]
  deg_out = dp[:, 0:1] + dp[:, 2:3]
  deg_in = dp[:, 1:2] + dp[:, 3:4]
  ns = lax.rsqrt(jnp.maximum(deg_out, 1.0))
  nd = lax.rsqrt(jnp.maximum(deg_in, 1.0))
  h1_ref[...] = jnp.dot(x_ref[...] * ns, w1_ref[...],
                        preferred_element_type=jnp.float32)
  norms_ref[...] = jnp.concatenate([ns, nd], axis=1)


def _tc2_body(aggp_ref, norms_ref, b1_ref, w2_ref, h2_ref):
  nrm = norms_ref[...]
  o1 = jnp.maximum((aggp_ref[0] + aggp_ref[1]) * nrm[:, 1:2] + b1_ref[...],
                   0.0)
  h2_ref[...] = jnp.dot(o1 * nrm[:, 0:1], w2_ref[...],
                        preferred_element_type=jnp.float32)


def _tc3_body(qp_ref, norms_ref, b2_ref, out_ref):
  nrm = norms_ref[...]
  q = (qp_ref[0] + qp_ref[1]) * nrm[:, 1:2]
  out_ref[...] = q[:, 0:C] + b2_ref[...]


@jax.jit
def kernel(x, edge_index, W1, b1, W2, b2):
  degp = _deg_kernel(edge_index)
  degp4 = degp.reshape(NC * 2, N).transpose(1, 0)

  h1, norms = pl.pallas_call(
      _tc1_body,
      out_shape=[
          jax.ShapeDtypeStruct((N, H), jnp.float32),
          jax.ShapeDtypeStruct((N, 2), jnp.float32),
      ],
  )(x, degp4, W1)

  aggp = _agg128(h1, edge_index)

  w2p = jnp.pad(W2, ((0, 0), (0, 16 - C)))
  h2 = pl.pallas_call(
      _tc2_body,
      out_shape=jax.ShapeDtypeStruct((N, 16), jnp.float32),
  )(aggp, norms, b1.reshape(1, H), w2p)

  qp = _agg16(h2, edge_index)

  out = pl.pallas_call(
      _tc3_body,
      out_shape=jax.ShapeDtypeStruct((N, C), jnp.float32),
  )(qp, norms, b2.reshape(1, C))
  return out
